# baseline (speedup 1.0000x reference)
import jax
from jax import lax
from jax.experimental import pallas as pl
from jax.experimental.pallas import tpu as pltpu
from jax.experimental.pallas import tpu_sc as plsc

_TC_GRID = 10
_E_ROWS = 320000
_E_COLS = 16
_NW = 32
_ROWS_PER_W = _E_ROWS // _NW
_CHUNK = 400
_NCHUNK = _ROWS_PER_W // _CHUNK


def _tc_body(n_in, ei_in, u_in, b_in, n_out, ei_out, u_out, b_out,
             u_sem, b_sem):
    i = pl.program_id(0)

    @pl.when(i == 0)
    def _start_small():
        pltpu.make_async_copy(u_in, u_out, u_sem).start()
        pltpu.make_async_copy(b_in, b_out, b_sem).start()

    n_out[...] = n_in[...]
    ei_out[...] = ei_in[...]

    @pl.when(i == pl.num_programs(0) - 1)
    def _wait_small():
        pltpu.make_async_copy(u_in, u_out, u_sem).wait()
        pltpu.make_async_copy(b_in, b_out, b_sem).wait()


def _sc_body(e_in, e_out, buf0, buf1, gs0, gs1, ss0, ss1):
    wid = lax.axis_index("s") * 2 + lax.axis_index("c")
    base = wid * _ROWS_PER_W
    bufs = (buf0, buf1)
    gsems = (gs0, gs1)
    ssems = (ss0, ss1)

    def _in(k):
        return e_in.at[pl.ds(base + k * _CHUNK, _CHUNK)]

    def _out(k):
        return e_out.at[pl.ds(base + k * _CHUNK, _CHUNK)]

    g_cur = pltpu.async_copy(_in(0), bufs[0], gsems[0])
    s_prev = None
    for k in range(_NCHUNK):
        b = k % 2
        g_cur.wait()
        s_cur = pltpu.async_copy(bufs[b], _out(k), ssems[b])
        if k + 1 < _NCHUNK:
            nb = (k + 1) % 2
            if s_prev is not None:
                s_prev.wait()
            g_cur = pltpu.async_copy(_in(k + 1), bufs[nb], gsems[nb])
        s_prev = s_cur
    s_prev.wait()


def _sc_edges_copy(edges):
    mesh = plsc.VectorSubcoreMesh(core_axis_name="c", subcore_axis_name="s")
    return pl.kernel(
        _sc_body,
        out_type=jax.ShapeDtypeStruct(edges.shape, edges.dtype),
        mesh=mesh,
        scratch_types=[
            pltpu.VMEM((_CHUNK, _E_COLS), edges.dtype),
            pltpu.VMEM((_CHUNK, _E_COLS), edges.dtype),
            pltpu.SemaphoreType.DMA,
            pltpu.SemaphoreType.DMA,
            pltpu.SemaphoreType.DMA,
            pltpu.SemaphoreType.DMA,
        ],
        compiler_params=pltpu.CompilerParams(use_tc_tiling_on_sc=True),
    )(edges)


def kernel(nodes, edge_index, edges, u, batch):
    g = _TC_GRID
    any_spec = pl.BlockSpec(memory_space=pl.ANY)
    specs = [
        pl.BlockSpec((nodes.shape[0] // g, nodes.shape[1]), lambda i: (i, 0)),
        pl.BlockSpec((edge_index.shape[0], edge_index.shape[1] // g),
                     lambda i: (0, i)),
        any_spec,
        any_spec,
    ]
    e_out = _sc_edges_copy(edges)
    out = pl.pallas_call(
        _tc_body,
        grid=(g,),
        in_specs=specs,
        out_specs=specs,
        out_shape=[
            jax.ShapeDtypeStruct(nodes.shape, nodes.dtype),
            jax.ShapeDtypeStruct(edge_index.shape, edge_index.dtype),
            jax.ShapeDtypeStruct(u.shape, u.dtype),
            jax.ShapeDtypeStruct(batch.shape, batch.dtype),
        ],
        scratch_shapes=[pltpu.SemaphoreType.DMA, pltpu.SemaphoreType.DMA],
    )(nodes, edge_index, u, batch)
    return (out[0], out[1], e_out, out[2], out[3])


# speedup vs baseline: 12.3125x; 12.3125x over previous
import jax
from jax.experimental import pallas as pl
from jax.experimental.pallas import tpu as pltpu

_GRID = 10


def _copy_body(n_in, ei_in, u_in, b_in,
               n_out, ei_out, u_out, b_out,
               u_sem, b_sem):
    i = pl.program_id(0)

    @pl.when(i == 0)
    def _start_small():
        pltpu.make_async_copy(u_in, u_out, u_sem).start()
        pltpu.make_async_copy(b_in, b_out, b_sem).start()

    n_out[...] = n_in[...]
    ei_out[...] = ei_in[...]

    @pl.when(i == pl.num_programs(0) - 1)
    def _wait_small():
        pltpu.make_async_copy(u_in, u_out, u_sem).wait()
        pltpu.make_async_copy(b_in, b_out, b_sem).wait()


def kernel(nodes, edge_index, edges, u, batch):
    g = _GRID
    any_spec = pl.BlockSpec(memory_space=pl.ANY)
    specs = [
        pl.BlockSpec((nodes.shape[0] // g, nodes.shape[1]), lambda i: (i, 0)),
        pl.BlockSpec((edge_index.shape[0], edge_index.shape[1] // g),
                     lambda i: (0, i)),
        any_spec,
        any_spec,
    ]
    out = pl.pallas_call(
        _copy_body,
        grid=(g,),
        in_specs=specs,
        out_specs=specs,
        out_shape=[
            jax.ShapeDtypeStruct(nodes.shape, nodes.dtype),
            jax.ShapeDtypeStruct(edge_index.shape, edge_index.dtype),
            jax.ShapeDtypeStruct(u.shape, u.dtype),
            jax.ShapeDtypeStruct(batch.shape, batch.dtype),
        ],
        scratch_shapes=[pltpu.SemaphoreType.DMA, pltpu.SemaphoreType.DMA],
    )(nodes, edge_index, u, batch)
    return (out[0], out[1], edges, out[2], out[3])


# speedup vs baseline: 13.4129x; 1.0894x over previous
import jax
from jax.experimental import pallas as pl
from jax.experimental.pallas import tpu as pltpu

_GRID = 5


def _copy_body(n_in, ei_in, u_in, b_in,
               n_out, ei_out, u_out, b_out,
               u_sem, b_sem):
    i = pl.program_id(0)

    @pl.when(i == 0)
    def _start_small():
        pltpu.make_async_copy(u_in, u_out, u_sem).start()
        pltpu.make_async_copy(b_in, b_out, b_sem).start()

    n_out[...] = n_in[...]
    ei_out[...] = ei_in[...]

    @pl.when(i == pl.num_programs(0) - 1)
    def _wait_small():
        pltpu.make_async_copy(u_in, u_out, u_sem).wait()
        pltpu.make_async_copy(b_in, b_out, b_sem).wait()


def kernel(nodes, edge_index, edges, u, batch):
    g = _GRID
    any_spec = pl.BlockSpec(memory_space=pl.ANY)
    specs = [
        pl.BlockSpec((nodes.shape[0] // g, nodes.shape[1]), lambda i: (i, 0)),
        pl.BlockSpec((edge_index.shape[0], edge_index.shape[1] // g),
                     lambda i: (0, i)),
        any_spec,
        any_spec,
    ]
    out = pl.pallas_call(
        _copy_body,
        grid=(g,),
        in_specs=specs,
        out_specs=specs,
        out_shape=[
            jax.ShapeDtypeStruct(nodes.shape, nodes.dtype),
            jax.ShapeDtypeStruct(edge_index.shape, edge_index.dtype),
            jax.ShapeDtypeStruct(u.shape, u.dtype),
            jax.ShapeDtypeStruct(batch.shape, batch.dtype),
        ],
        scratch_shapes=[pltpu.SemaphoreType.DMA, pltpu.SemaphoreType.DMA],
    )(nodes, edge_index, u, batch)
    return (out[0], out[1], edges, out[2], out[3])


# speedup vs baseline: 14.0269x; 1.0458x over previous
import jax
from jax.experimental import pallas as pl
from jax.experimental.pallas import tpu as pltpu

_GRID = 2


def _copy_body(n_in, ei_in, u_in, b_in,
               n_out, ei_out, u_out, b_out,
               u_sem, b_sem):
    i = pl.program_id(0)

    @pl.when(i == 0)
    def _start_small():
        pltpu.make_async_copy(u_in, u_out, u_sem).start()
        pltpu.make_async_copy(b_in, b_out, b_sem).start()

    n_out[...] = n_in[...]
    ei_out[...] = ei_in[...]

    @pl.when(i == pl.num_programs(0) - 1)
    def _wait_small():
        pltpu.make_async_copy(u_in, u_out, u_sem).wait()
        pltpu.make_async_copy(b_in, b_out, b_sem).wait()


def kernel(nodes, edge_index, edges, u, batch):
    g = _GRID
    any_spec = pl.BlockSpec(memory_space=pl.ANY)
    specs = [
        pl.BlockSpec((nodes.shape[0] // g, nodes.shape[1]), lambda i: (i, 0)),
        pl.BlockSpec((edge_index.shape[0], edge_index.shape[1] // g),
                     lambda i: (0, i)),
        any_spec,
        any_spec,
    ]
    out = pl.pallas_call(
        _copy_body,
        grid=(g,),
        in_specs=specs,
        out_specs=specs,
        out_shape=[
            jax.ShapeDtypeStruct(nodes.shape, nodes.dtype),
            jax.ShapeDtypeStruct(edge_index.shape, edge_index.dtype),
            jax.ShapeDtypeStruct(u.shape, u.dtype),
            jax.ShapeDtypeStruct(batch.shape, batch.dtype),
        ],
        scratch_shapes=[pltpu.SemaphoreType.DMA, pltpu.SemaphoreType.DMA],
    )(nodes, edge_index, u, batch)
    return (out[0], out[1], edges, out[2], out[3])
